# Initial kernel scaffold; baseline (speedup 1.0000x reference)
import numpy as np
import jax
import jax.numpy as jnp
from jax import lax
from jax.experimental import pallas as pl
from jax.experimental.pallas import tpu as pltpu
from jax.experimental.pallas import tpu_sc as plsc

N_NODES = 10000
N_EDGES = 160000
NUM_ELEMENTS = 100
Z_DIM = 48
MOL_DIM = 16
NODE_DIM = 64
NUM_BASIS = 10
MID = 32
MAX_RADIUS = 2.0
MULS = [4, 2, 1, 1]
DIMS = [1, 3, 5, 7]
NPATH = 8
OUT_DIM = 22
EVW = 32
TW = 80
PW = 16

NC, NS = 2, 16
NW = NC * NS
EPW = N_EDGES // NW
CH = 125
NCHUNK = EPW // CH
RPT = N_NODES // NS

_t = np.linspace(-12.0, 12.0, 480001)
_pdf = np.exp(-0.5 * _t * _t) / np.sqrt(2.0 * np.pi)
_s = _t / (1.0 + np.exp(-_t))
_trapz = getattr(np, "trapz", None) or np.trapezoid
SILU_CST = float(1.0 / np.sqrt(_trapz(_s * _s * _pdf, _t)))

STEP = MAX_RADIUS / (NUM_BASIS + 1)
EMB_CST = 1.14136 * float(np.exp(2.0))

_OFFS = [0, 256, 384, 448]
_PATH = []
for _l, _m in enumerate(MULS):
    for _j in range(_m):
        _PATH.append((_l, _OFFS[_l], _m, _j))

_COLIDX = np.zeros((NODE_DIM, NPATH), dtype=np.int32)
for _p, (_l, _off, _m, _j) in enumerate(_PATH):
    for _u in range(NODE_DIM):
        _COLIDX[_u, _p] = _off + _u * _m + _j

_SH_OFF = [0, 1, 4, 9]
_O2P = np.zeros(OUT_DIM, dtype=np.int32)
_O2D = np.zeros(OUT_DIM, dtype=np.int32)
_o = 0
for _p, (_l, _off, _m, _j) in enumerate(_PATH):
    for _c in range(DIMS[_l]):
        _O2P[_o] = _p
        _O2D[_o] = _SH_OFF[_l] + _c
        _o += 1

_PC = np.zeros((NPATH, EVW), dtype=np.float32)
_PS = np.zeros((16, EVW), dtype=np.float32)
for _o in range(OUT_DIM):
    _PC[_O2P[_o], _o] = 1.0
    _PS[_O2D[_o], _o] = 1.0

_TILE8 = np.tile(np.eye(MID, dtype=np.float32), (1, NPATH))
_SUMR = np.zeros((MID * NPATH, NPATH), dtype=np.float32)
for _p in range(NPATH):
    _SUMR[_p * MID:(_p + 1) * MID, _p] = 1.0

_S3, _S15, _S5 = np.sqrt(3.0), np.sqrt(15.0), np.sqrt(5.0)
_S105, _S7 = np.sqrt(105.0), np.sqrt(7.0)
_S35_8, _S21_8 = np.sqrt(35.0 / 8.0), np.sqrt(21.0 / 8.0)


def _table_body(x_ref, mol_ref, pos_ref, zt_ref, mt_ref, t_ref, p_ref):
    nb = x_ref.shape[0]
    ioz = lax.broadcasted_iota(jnp.int32, (nb, 128), 1)
    oh = (x_ref[...] == ioz).astype(jnp.float32)
    iom = lax.broadcasted_iota(jnp.int32, (nb, 8), 1)
    ohm = (mol_ref[...] == iom).astype(jnp.float32)
    h = (jnp.dot(oh, zt_ref[...], preferred_element_type=jnp.float32)
         + jnp.dot(ohm, mt_ref[...], preferred_element_type=jnp.float32))
    pos = pos_ref[...]
    zpad = jnp.zeros((nb, TW - 67), dtype=jnp.float32)
    t_ref[...] = jnp.concatenate([h, pos, zpad], axis=1)
    p_ref[...] = jnp.concatenate([pos, jnp.zeros((nb, PW - 3), jnp.float32)],
                                 axis=1)


def _gather_body(t_hbm, p_hbm, src_hbm, dst_hbm, g1_hbm, g2_hbm,
                 idxs_v, idxd_v, buft_v, bufp_v, semt, semp):
    wid = lax.axis_index("c") * NS + lax.axis_index("s")
    base = wid * EPW
    pltpu.sync_copy(src_hbm.at[wid], idxs_v)
    pltpu.sync_copy(dst_hbm.at[wid], idxd_v)

    def chunk(j, carry):
        ct = pltpu.async_copy(t_hbm.at[idxs_v.at[j]], buft_v, semt)
        cp = pltpu.async_copy(p_hbm.at[idxd_v.at[j]], bufp_v, semp)
        ct.wait()
        cp.wait()
        pltpu.sync_copy(buft_v, g1_hbm.at[pl.ds(base + j * CH, CH)])
        pltpu.sync_copy(bufp_v, g2_hbm.at[pl.ds(base + j * CH, CH)])
        return carry

    lax.fori_loop(0, NCHUNK, chunk, 0)


def _edge_body(g1_ref, g2_ref, w1_ref, w2_ref, t8_ref, sr_ref, pc_ref, ps_ref,
               ev_ref):
    eb = g1_ref.shape[0]
    h_src = g1_ref[:, :NODE_DIM]
    vec = g2_ref[:, 0:3] - g1_ref[:, NODE_DIM:NODE_DIM + 3]
    elen = jnp.sqrt(jnp.sum(vec * vec, axis=1, keepdims=True))
    unit = vec / (elen + 1e-12)
    ux, uy, uz = unit[:, 0:1], unit[:, 1:2], unit[:, 2:3]

    xx, yy, zz = ux * ux, uy * uy, uz * uz
    one = jnp.ones_like(ux)
    sh = jnp.concatenate([
        one,
        _S3 * ux, _S3 * uy, _S3 * uz,
        _S15 * ux * uy,
        _S15 * uy * uz,
        (_S5 / 2.0) * (3.0 * zz - 1.0),
        _S15 * ux * uz,
        (_S15 / 2.0) * (xx - yy),
        _S35_8 * uy * (3.0 * xx - yy),
        _S105 * ux * uy * uz,
        _S21_8 * uy * (5.0 * zz - 1.0),
        (_S7 / 2.0) * uz * (5.0 * zz - 3.0),
        _S21_8 * ux * (5.0 * zz - 1.0),
        (_S105 / 2.0) * uz * (xx - yy),
        _S35_8 * ux * (xx - 3.0 * yy),
    ], axis=1)

    iob = lax.broadcasted_iota(jnp.float32, (eb, 16), 1)
    dif = elen / STEP - iob - 1.0

    def sus(t):
        return jnp.where(t > 0, jnp.exp(-1.0 / jnp.where(t > 0, t, 1.0)), 0.0)

    f = sus(dif + 1.0) * sus(1.0 - dif)
    lane = lax.broadcasted_iota(jnp.int32, (eb, 16), 1)
    f = jnp.where(lane < NUM_BASIS, f, 0.0)

    z1 = jnp.dot(f, w1_ref[...], preferred_element_type=jnp.float32)
    h1 = (z1 / (1.0 + jnp.exp(-z1))) * SILU_CST

    a = jnp.dot(h_src, w2_ref[...], preferred_element_type=jnp.float32)
    htile = jnp.dot(h1, t8_ref[...], preferred_element_type=jnp.float32)
    coeff = jnp.dot(a * htile, sr_ref[...],
                    preferred_element_type=jnp.float32)
    ev_ref[...] = (jnp.dot(coeff, pc_ref[...],
                           preferred_element_type=jnp.float32)
                   * jnp.dot(sh, ps_ref[...],
                             preferred_element_type=jnp.float32))


def _scatter_body(ev_hbm, dst_hbm, zero_hbm, nv_hbm,
                  idxd_v, evbuf_v, zbuf_v, acc_sh, dsem):
    cid = lax.axis_index("c")
    sid = lax.axis_index("s")
    wid = cid * NS + sid
    base = wid * EPW

    pltpu.sync_copy(zero_hbm.at[pl.ds(sid * RPT, RPT)], zbuf_v)
    pltpu.sync_copy(zbuf_v, acc_sh.at[pl.ds(sid * RPT, RPT)])
    plsc.subcore_barrier()

    pltpu.sync_copy(dst_hbm.at[wid], idxd_v)

    def chunk(j, carry):
        pltpu.sync_copy(ev_hbm.at[pl.ds(base + j * CH, CH)], evbuf_v)
        pltpu.sync_copy(evbuf_v, acc_sh.at[idxd_v.at[j]], add=True)
        return carry

    lax.fori_loop(0, NCHUNK, chunk, 0)
    plsc.subcore_barrier()

    pltpu.sync_copy(acc_sh.at[pl.ds(sid * RPT, RPT)], zbuf_v)
    pltpu.sync_copy(zbuf_v, nv_hbm.at[cid, pl.ds(sid * RPT, RPT)])


def _ln_body(nv_ref, g_ref, b_ref, out_ref):
    nb = nv_ref.shape[1]
    v = nv_ref[0] + nv_ref[1]
    mu = jnp.sum(v, axis=1, keepdims=True) * (1.0 / OUT_DIM)
    lane = lax.broadcasted_iota(jnp.int32, (nb, EVW), 1)
    d = jnp.where(lane < OUT_DIM, v - mu, 0.0)
    var = jnp.sum(d * d, axis=1, keepdims=True) * (1.0 / OUT_DIM)
    res = d * lax.rsqrt(var + 1e-5) * g_ref[...] + b_ref[...]
    out_ref[...] = res[:, :OUT_DIM]


@jax.jit
def _run(x, pos, mol_id, edge_src, edge_dst, zt_pad, mt_pad, w1p, w2rt,
         t8, sr, pc, ps, gamma_p, beta_p, zeros_nv):
    nb = 1000
    grid_a = N_NODES // nb
    tbl, ptbl = pl.pallas_call(
        _table_body,
        grid=(grid_a,),
        in_specs=[
            pl.BlockSpec((nb, 1), lambda i: (i, 0)),
            pl.BlockSpec((nb, 1), lambda i: (i, 0)),
            pl.BlockSpec((nb, 3), lambda i: (i, 0)),
            pl.BlockSpec((128, NODE_DIM), lambda i: (0, 0)),
            pl.BlockSpec((8, NODE_DIM), lambda i: (0, 0)),
        ],
        out_specs=[
            pl.BlockSpec((nb, TW), lambda i: (i, 0)),
            pl.BlockSpec((nb, PW), lambda i: (i, 0)),
        ],
        out_shape=[
            jax.ShapeDtypeStruct((N_NODES, TW), jnp.float32),
            jax.ShapeDtypeStruct((N_NODES, PW), jnp.float32),
        ],
    )(x.reshape(N_NODES, 1), mol_id.reshape(N_NODES, 1), pos, zt_pad, mt_pad)

    src3 = edge_src.reshape(NW, NCHUNK, CH)
    dst3 = edge_dst.reshape(NW, NCHUNK, CH)

    g1, g2 = pl.kernel(
        _gather_body,
        out_type=[
            jax.ShapeDtypeStruct((N_EDGES, TW), jnp.float32),
            jax.ShapeDtypeStruct((N_EDGES, PW), jnp.float32),
        ],
        mesh=plsc.VectorSubcoreMesh(core_axis_name="c", subcore_axis_name="s"),
        scratch_types=[
            pltpu.VMEM((NCHUNK, CH), jnp.int32),
            pltpu.VMEM((NCHUNK, CH), jnp.int32),
            pltpu.VMEM((CH, TW), jnp.float32),
            pltpu.VMEM((CH, PW), jnp.float32),
            pltpu.SemaphoreType.DMA,
            pltpu.SemaphoreType.DMA,
        ],
    )(tbl, ptbl, src3, dst3)

    eb = 640
    grid_c = N_EDGES // eb
    ev = pl.pallas_call(
        _edge_body,
        grid=(grid_c,),
        in_specs=[
            pl.BlockSpec((eb, TW), lambda i: (i, 0)),
            pl.BlockSpec((eb, PW), lambda i: (i, 0)),
            pl.BlockSpec((16, MID), lambda i: (0, 0)),
            pl.BlockSpec((NODE_DIM, MID * NPATH), lambda i: (0, 0)),
            pl.BlockSpec((MID, MID * NPATH), lambda i: (0, 0)),
            pl.BlockSpec((MID * NPATH, NPATH), lambda i: (0, 0)),
            pl.BlockSpec((NPATH, EVW), lambda i: (0, 0)),
            pl.BlockSpec((16, EVW), lambda i: (0, 0)),
        ],
        out_specs=pl.BlockSpec((eb, EVW), lambda i: (i, 0)),
        out_shape=jax.ShapeDtypeStruct((N_EDGES, EVW), jnp.float32),
    )(g1, g2, w1p, w2rt, t8, sr, pc, ps)

    nv2 = pl.kernel(
        _scatter_body,
        out_type=jax.ShapeDtypeStruct((NC, N_NODES, EVW), jnp.float32),
        mesh=plsc.VectorSubcoreMesh(core_axis_name="c", subcore_axis_name="s"),
        scratch_types=[
            pltpu.VMEM((NCHUNK, CH), jnp.int32),
            pltpu.VMEM((CH, EVW), jnp.float32),
            pltpu.VMEM((RPT, EVW), jnp.float32),
            pltpu.VMEM_SHARED((N_NODES, EVW), jnp.float32),
            pltpu.SemaphoreType.DMA,
        ],
    )(ev, dst3, zeros_nv)

    out = pl.pallas_call(
        _ln_body,
        grid=(grid_a,),
        in_specs=[
            pl.BlockSpec((NC, nb, EVW), lambda i: (0, i, 0)),
            pl.BlockSpec((1, EVW), lambda i: (0, 0)),
            pl.BlockSpec((1, EVW), lambda i: (0, 0)),
        ],
        out_specs=pl.BlockSpec((nb, OUT_DIM), lambda i: (i, 0)),
        out_shape=jax.ShapeDtypeStruct((N_NODES, OUT_DIM), jnp.float32),
    )(nv2, gamma_p, beta_p)
    return out


def kernel(x, pos, mol_id, edge_src, edge_dst, z_table, mol_table, W1, W2,
           gamma, beta):
    x = x.astype(jnp.int32)
    mol_id = mol_id.astype(jnp.int32)
    edge_src = edge_src.astype(jnp.int32)
    edge_dst = edge_dst.astype(jnp.int32)

    zt_pad = jnp.zeros((128, NODE_DIM), jnp.float32)
    zt_pad = zt_pad.at[:NUM_ELEMENTS, :Z_DIM].set(z_table)
    mt_pad = jnp.zeros((8, NODE_DIM), jnp.float32)
    mt_pad = mt_pad.at[:2, Z_DIM:].set(mol_table)

    w1p = jnp.zeros((16, MID), jnp.float32).at[:NUM_BASIS].set(W1 * EMB_CST)

    scale = 1.0 / (np.sqrt(float(MID)) * np.sqrt(float(NODE_DIM))
                   * np.sqrt(float(N_EDGES) / float(N_NODES)))
    w2rt = W2.T[_COLIDX.reshape(-1)].reshape(NODE_DIM, NPATH, MID)
    w2rt = (w2rt * scale).reshape(NODE_DIM, NPATH * MID)

    gamma_p = jnp.zeros((1, EVW), jnp.float32).at[0, :OUT_DIM].set(gamma)
    beta_p = jnp.zeros((1, EVW), jnp.float32).at[0, :OUT_DIM].set(beta)
    zeros_nv = jnp.zeros((N_NODES, EVW), jnp.float32)

    return _run(x, pos, mol_id, edge_src, edge_dst, zt_pad, mt_pad, w1p,
                w2rt, jnp.asarray(_TILE8), jnp.asarray(_SUMR),
                jnp.asarray(_PC), jnp.asarray(_PS), gamma_p, beta_p,
                zeros_nv)


# speedup vs baseline: 1.2374x; 1.2374x over previous
import numpy as np
import jax
import jax.numpy as jnp
from jax import lax
from jax.experimental import pallas as pl
from jax.experimental.pallas import tpu as pltpu
from jax.experimental.pallas import tpu_sc as plsc

N_NODES = 10000
N_EDGES = 160000
NUM_ELEMENTS = 100
Z_DIM = 48
MOL_DIM = 16
NODE_DIM = 64
NUM_BASIS = 10
MID = 32
MAX_RADIUS = 2.0
MULS = [4, 2, 1, 1]
DIMS = [1, 3, 5, 7]
NPATH = 8
OUT_DIM = 22
EVW = 32
TW = 128

NC, NS = 2, 16
NW = NC * NS
EPW = N_EDGES // NW
CH = 128
NCHUNK = 40
EPWP = CH * NCHUNK
E_PAD = EPWP * NW
NEG = 8
NCG = 4
ECG = E_PAD // NEG
NCH2 = ECG // CH
NPAD = 10112
RPT = NPAD // NS

_t = np.linspace(-12.0, 12.0, 480001)
_pdf = np.exp(-0.5 * _t * _t) / np.sqrt(2.0 * np.pi)
_s = _t / (1.0 + np.exp(-_t))
_trapz = getattr(np, "trapz", None) or np.trapezoid
SILU_CST = float(1.0 / np.sqrt(_trapz(_s * _s * _pdf, _t)))

STEP = MAX_RADIUS / (NUM_BASIS + 1)
EMB_CST = 1.14136 * float(np.exp(2.0))

_OFFS = [0, 256, 384, 448]
_PATH = []
for _l, _m in enumerate(MULS):
    for _j in range(_m):
        _PATH.append((_l, _OFFS[_l], _m, _j))

_COLIDX = np.zeros((NODE_DIM, NPATH), dtype=np.int32)
for _p, (_l, _off, _m, _j) in enumerate(_PATH):
    for _u in range(NODE_DIM):
        _COLIDX[_u, _p] = _off + _u * _m + _j

_SH_OFF = [0, 1, 4, 9]
_O2P = np.zeros(OUT_DIM, dtype=np.int32)
_O2D = np.zeros(OUT_DIM, dtype=np.int32)
_o = 0
for _p, (_l, _off, _m, _j) in enumerate(_PATH):
    for _c in range(DIMS[_l]):
        _O2P[_o] = _p
        _O2D[_o] = _SH_OFF[_l] + _c
        _o += 1

_PC = np.zeros((NPATH, EVW), dtype=np.float32)
_PS = np.zeros((16, EVW), dtype=np.float32)
for _o in range(OUT_DIM):
    _PC[_O2P[_o], _o] = 1.0
    _PS[_O2D[_o], _o] = 1.0

_TILE8 = np.tile(np.eye(MID, dtype=np.float32), (1, NPATH))
_SUMR = np.zeros((MID * NPATH, NPATH), dtype=np.float32)
for _p in range(NPATH):
    _SUMR[_p * MID:(_p + 1) * MID, _p] = 1.0

_S3, _S15, _S5 = np.sqrt(3.0), np.sqrt(15.0), np.sqrt(5.0)
_S105, _S7 = np.sqrt(105.0), np.sqrt(7.0)
_S35_8, _S21_8 = np.sqrt(35.0 / 8.0), np.sqrt(21.0 / 8.0)


def _table_body(x_ref, mol_ref, pos_ref, zt_ref, mt_ref, t_ref):
    nb = x_ref.shape[0]
    ioz = lax.broadcasted_iota(jnp.int32, (nb, 128), 1)
    oh = (x_ref[...] == ioz).astype(jnp.float32)
    iom = lax.broadcasted_iota(jnp.int32, (nb, 8), 1)
    ohm = (mol_ref[...] == iom).astype(jnp.float32)
    h = (jnp.dot(oh, zt_ref[...], preferred_element_type=jnp.float32)
         + jnp.dot(ohm, mt_ref[...], preferred_element_type=jnp.float32))
    pos = pos_ref[...]
    zpad = jnp.zeros((nb, TW - 67), dtype=jnp.float32)
    t_ref[...] = jnp.concatenate([h, pos, zpad], axis=1)


def _gather_body(t_hbm, src_hbm, dst_hbm, g1_hbm, g2_hbm,
                 idxs_v, idxd_v, bufs_v, bufd_v, sems, semd):
    sid = lax.axis_index("s")
    wid = lax.axis_index("c") * NS + sid
    base = wid * EPWP
    pltpu.sync_copy(src_hbm.at[wid], idxs_v)
    pltpu.sync_copy(dst_hbm.at[wid], idxd_v)

    def chunk(j, carry):
        cs = pltpu.async_copy(t_hbm.at[idxs_v.at[j]], bufs_v, sems)
        cd = pltpu.async_copy(t_hbm.at[idxd_v.at[j]], bufd_v, semd)
        cs.wait()
        cd.wait()
        pltpu.sync_copy(bufs_v, g1_hbm.at[pl.ds(base + j * CH, CH)])
        pltpu.sync_copy(bufd_v, g2_hbm.at[pl.ds(base + j * CH, CH)])
        return carry

    lax.fori_loop(0, NCHUNK, chunk, 0)


def _edge_body(g1_ref, g2_ref, w1_ref, w2_ref, t8_ref, sr_ref, pc_ref, ps_ref,
               ev_ref):
    eb = g1_ref.shape[0]
    h_src = g1_ref[:, :NODE_DIM]
    vec = g2_ref[:, 64:67] - g1_ref[:, 64:67]
    elen = jnp.sqrt(jnp.sum(vec * vec, axis=1, keepdims=True))
    unit = vec / (elen + 1e-12)
    ux, uy, uz = unit[:, 0:1], unit[:, 1:2], unit[:, 2:3]

    xx, yy, zz = ux * ux, uy * uy, uz * uz
    one = jnp.ones_like(ux)
    sh = jnp.concatenate([
        one,
        _S3 * ux, _S3 * uy, _S3 * uz,
        _S15 * ux * uy,
        _S15 * uy * uz,
        (_S5 / 2.0) * (3.0 * zz - 1.0),
        _S15 * ux * uz,
        (_S15 / 2.0) * (xx - yy),
        _S35_8 * uy * (3.0 * xx - yy),
        _S105 * ux * uy * uz,
        _S21_8 * uy * (5.0 * zz - 1.0),
        (_S7 / 2.0) * uz * (5.0 * zz - 3.0),
        _S21_8 * ux * (5.0 * zz - 1.0),
        (_S105 / 2.0) * uz * (xx - yy),
        _S35_8 * ux * (xx - 3.0 * yy),
    ], axis=1)

    lane = lax.broadcasted_iota(jnp.int32, (eb, 16), 1)
    dif = elen / STEP - lane.astype(jnp.float32) - 1.0

    def sus(t):
        return jnp.where(t > 0, jnp.exp(-1.0 / jnp.where(t > 0, t, 1.0)), 0.0)

    f = sus(dif + 1.0) * sus(1.0 - dif)
    f = jnp.where(lane < NUM_BASIS, f, 0.0)

    z1 = jnp.dot(f, w1_ref[...], preferred_element_type=jnp.float32)
    h1 = (z1 / (1.0 + jnp.exp(-z1))) * SILU_CST

    a = jnp.dot(h_src, w2_ref[...], preferred_element_type=jnp.float32)
    htile = jnp.dot(h1, t8_ref[...], preferred_element_type=jnp.float32)
    coeff = jnp.dot(a * htile, sr_ref[...],
                    preferred_element_type=jnp.float32)
    ev = (jnp.dot(coeff, pc_ref[...], preferred_element_type=jnp.float32)
          * jnp.dot(sh, ps_ref[...], preferred_element_type=jnp.float32))
    row = (pl.program_id(0) * eb
           + lax.broadcasted_iota(jnp.int32, (eb, 1), 0))
    valid = jnp.mod(row, EPWP) < EPW
    ev = jnp.where(valid, ev, 0.0)
    zpad8 = jnp.zeros((eb, 8), jnp.float32)
    for cg in range(NCG):
        ev_ref[cg] = jnp.concatenate([ev[:, cg * 8:(cg + 1) * 8], zpad8],
                                     axis=1)


def _scatter_body(ev_hbm, idx_hbm, zero_hbm, nv_hbm, idxb_v, evb_v, acc_v):
    wid = lax.axis_index("c") * NS + lax.axis_index("s")
    cg = wid // NEG
    eg = wid % NEG
    ebase = cg * E_PAD + eg * ECG

    pltpu.sync_copy(zero_hbm, acc_v)

    def chunk(j, carry):
        pltpu.sync_copy(ev_hbm.at[pl.ds(ebase + j * CH, CH)], evb_v)
        pltpu.sync_copy(idx_hbm.at[pl.ds(eg * ECG + j * CH, CH)], idxb_v)

        def row(r, c2):
            plsc.addupdate_scatter(acc_v, [idxb_v[r]], evb_v[r])
            return c2

        lax.fori_loop(0, CH, row, 0)
        return carry

    lax.fori_loop(0, NCH2, chunk, 0)
    pltpu.sync_copy(acc_v, nv_hbm.at[pl.ds(wid * NPAD * 8, NPAD * 8)])


def _ln_body(nv_ref, g_ref, b_ref, out_ref):
    nb = nv_ref.shape[1]
    parts = []
    for cg in range(NCG):
        p = nv_ref[cg * NEG]
        for eg in range(1, NEG):
            p = p + nv_ref[cg * NEG + eg]
        parts.append(p)
    v = jnp.concatenate(parts, axis=1)
    mu = jnp.sum(v, axis=1, keepdims=True) * (1.0 / OUT_DIM)
    lane = lax.broadcasted_iota(jnp.int32, (nb, EVW), 1)
    d = jnp.where(lane < OUT_DIM, v - mu, 0.0)
    var = jnp.sum(d * d, axis=1, keepdims=True) * (1.0 / OUT_DIM)
    res = d * lax.rsqrt(var + 1e-5) * g_ref[...] + b_ref[...]
    out_ref[...] = res[:, :OUT_DIM]


@jax.jit
def _run(x, pos, mol_id, edge_src, edge_dst, zt_pad, mt_pad, w1p, w2rt,
         t8, sr, pc, ps, gamma_p, beta_p, zeros_nv):
    nb = 1000
    grid_a = N_NODES // nb
    tbl = pl.pallas_call(
        _table_body,
        grid=(grid_a,),
        in_specs=[
            pl.BlockSpec((nb, 1), lambda i: (i, 0)),
            pl.BlockSpec((nb, 1), lambda i: (i, 0)),
            pl.BlockSpec((nb, 3), lambda i: (i, 0)),
            pl.BlockSpec((128, NODE_DIM), lambda i: (0, 0)),
            pl.BlockSpec((8, NODE_DIM), lambda i: (0, 0)),
        ],
        out_specs=pl.BlockSpec((nb, TW), lambda i: (i, 0)),
        out_shape=jax.ShapeDtypeStruct((N_NODES, TW), jnp.float32),
    )(x.reshape(N_NODES, 1), mol_id.reshape(N_NODES, 1), pos, zt_pad, mt_pad)

    src3 = jnp.pad(edge_src.reshape(NW, EPW),
                   ((0, 0), (0, EPWP - EPW))).reshape(NW, NCHUNK, CH)
    dst3 = jnp.pad(edge_dst.reshape(NW, EPW),
                   ((0, 0), (0, EPWP - EPW))).reshape(NW, NCHUNK, CH)

    g1, g2 = pl.kernel(
        _gather_body,
        out_type=[
            jax.ShapeDtypeStruct((E_PAD, TW), jnp.float32),
            jax.ShapeDtypeStruct((E_PAD, TW), jnp.float32),
        ],
        mesh=plsc.VectorSubcoreMesh(core_axis_name="c", subcore_axis_name="s"),
        scratch_types=[
            pltpu.VMEM((NCHUNK, CH), jnp.int32),
            pltpu.VMEM((NCHUNK, CH), jnp.int32),
            pltpu.VMEM((CH, TW), jnp.float32),
            pltpu.VMEM((CH, TW), jnp.float32),
            pltpu.SemaphoreType.DMA,
            pltpu.SemaphoreType.DMA,
        ],
    )(tbl, src3, dst3)

    eb = 640
    grid_c = E_PAD // eb
    ev = pl.pallas_call(
        _edge_body,
        grid=(grid_c,),
        in_specs=[
            pl.BlockSpec((eb, TW), lambda i: (i, 0)),
            pl.BlockSpec((eb, TW), lambda i: (i, 0)),
            pl.BlockSpec((16, MID), lambda i: (0, 0)),
            pl.BlockSpec((NODE_DIM, MID * NPATH), lambda i: (0, 0)),
            pl.BlockSpec((MID, MID * NPATH), lambda i: (0, 0)),
            pl.BlockSpec((MID * NPATH, NPATH), lambda i: (0, 0)),
            pl.BlockSpec((NPATH, EVW), lambda i: (0, 0)),
            pl.BlockSpec((16, EVW), lambda i: (0, 0)),
        ],
        out_specs=pl.BlockSpec((NCG, eb, 16), lambda i: (0, i, 0)),
        out_shape=jax.ShapeDtypeStruct((NCG, E_PAD, 16), jnp.float32),
    )(g1, g2, w1p, w2rt, t8, sr, pc, ps)
    ev = ev.reshape(NCG * E_PAD, 16)

    dstp = jnp.pad(edge_dst.reshape(NW, EPW),
                   ((0, 0), (0, EPWP - EPW))).reshape(E_PAD, 1)
    cpat = jnp.minimum(jnp.arange(16, dtype=jnp.int32), 7)[None, :]
    idxe = dstp * 8 + cpat
    nv2 = pl.kernel(
        _scatter_body,
        out_type=jax.ShapeDtypeStruct((NW * NPAD * 8,), jnp.float32),
        mesh=plsc.VectorSubcoreMesh(core_axis_name="c", subcore_axis_name="s"),
        compiler_params=pltpu.CompilerParams(needs_layout_passes=False),
        scratch_types=[
            pltpu.VMEM((CH, 16), jnp.int32),
            pltpu.VMEM((CH, 16), jnp.float32),
            pltpu.VMEM((NPAD * 8,), jnp.float32),
        ],
    )(ev, idxe, zeros_nv)
    nv2 = nv2.reshape(NW, NPAD, 8)

    out = pl.pallas_call(
        _ln_body,
        grid=(grid_a,),
        in_specs=[
            pl.BlockSpec((NW, nb, 8), lambda i: (0, i, 0)),
            pl.BlockSpec((1, EVW), lambda i: (0, 0)),
            pl.BlockSpec((1, EVW), lambda i: (0, 0)),
        ],
        out_specs=pl.BlockSpec((nb, OUT_DIM), lambda i: (i, 0)),
        out_shape=jax.ShapeDtypeStruct((N_NODES, OUT_DIM), jnp.float32),
    )(nv2, gamma_p, beta_p)
    return out


def kernel(x, pos, mol_id, edge_src, edge_dst, z_table, mol_table, W1, W2,
           gamma, beta):
    x = x.astype(jnp.int32)
    mol_id = mol_id.astype(jnp.int32)
    edge_src = edge_src.astype(jnp.int32)
    edge_dst = edge_dst.astype(jnp.int32)

    zt_pad = jnp.zeros((128, NODE_DIM), jnp.float32)
    zt_pad = zt_pad.at[:NUM_ELEMENTS, :Z_DIM].set(z_table)
    mt_pad = jnp.zeros((8, NODE_DIM), jnp.float32)
    mt_pad = mt_pad.at[:2, Z_DIM:].set(mol_table)

    w1p = jnp.zeros((16, MID), jnp.float32).at[:NUM_BASIS].set(W1 * EMB_CST)

    scale = 1.0 / (np.sqrt(float(MID)) * np.sqrt(float(NODE_DIM))
                   * np.sqrt(float(N_EDGES) / float(N_NODES)))
    w2rt = W2.T[_COLIDX.reshape(-1)].reshape(NODE_DIM, NPATH, MID)
    w2rt = (w2rt * scale).reshape(NODE_DIM, NPATH * MID)

    gamma_p = jnp.zeros((1, EVW), jnp.float32).at[0, :OUT_DIM].set(gamma)
    beta_p = jnp.zeros((1, EVW), jnp.float32).at[0, :OUT_DIM].set(beta)
    zeros_nv = jnp.zeros((NPAD * 8,), jnp.float32)

    return _run(x, pos, mol_id, edge_src, edge_dst, zt_pad, mt_pad, w1p,
                w2rt, jnp.asarray(_TILE8), jnp.asarray(_SUMR),
                jnp.asarray(_PC), jnp.asarray(_PS), gamma_p, beta_p,
                zeros_nv)


# speedup vs baseline: 1.3505x; 1.0914x over previous
import numpy as np
import jax
import jax.numpy as jnp
from jax import lax
from jax.experimental import pallas as pl
from jax.experimental.pallas import tpu as pltpu
from jax.experimental.pallas import tpu_sc as plsc

N_NODES = 10000
N_EDGES = 160000
NUM_ELEMENTS = 100
Z_DIM = 48
MOL_DIM = 16
NODE_DIM = 64
NUM_BASIS = 10
MID = 32
MAX_RADIUS = 2.0
MULS = [4, 2, 1, 1]
DIMS = [1, 3, 5, 7]
NPATH = 8
OUT_DIM = 22
EVW = 32
TW = 128

NC, NS = 2, 16
NW = NC * NS
EPW = N_EDGES // NW
CH = 128
NCHUNK = 40
EPWP = CH * NCHUNK
E_PAD = EPWP * NW
NEG = 8
NCG = 4
ECG = E_PAD // NEG
NCH2 = ECG // CH
NPAD = 10112
RPT = NPAD // NS

_t = np.linspace(-12.0, 12.0, 480001)
_pdf = np.exp(-0.5 * _t * _t) / np.sqrt(2.0 * np.pi)
_s = _t / (1.0 + np.exp(-_t))
_trapz = getattr(np, "trapz", None) or np.trapezoid
SILU_CST = float(1.0 / np.sqrt(_trapz(_s * _s * _pdf, _t)))

STEP = MAX_RADIUS / (NUM_BASIS + 1)
EMB_CST = 1.14136 * float(np.exp(2.0))

_OFFS = [0, 256, 384, 448]
_PATH = []
for _l, _m in enumerate(MULS):
    for _j in range(_m):
        _PATH.append((_l, _OFFS[_l], _m, _j))

_COLIDX = np.zeros((NODE_DIM, NPATH), dtype=np.int32)
for _p, (_l, _off, _m, _j) in enumerate(_PATH):
    for _u in range(NODE_DIM):
        _COLIDX[_u, _p] = _off + _u * _m + _j

_SH_OFF = [0, 1, 4, 9]
_O2P = np.zeros(OUT_DIM, dtype=np.int32)
_O2D = np.zeros(OUT_DIM, dtype=np.int32)
_o = 0
for _p, (_l, _off, _m, _j) in enumerate(_PATH):
    for _c in range(DIMS[_l]):
        _O2P[_o] = _p
        _O2D[_o] = _SH_OFF[_l] + _c
        _o += 1

_PC = np.zeros((NPATH, EVW), dtype=np.float32)
_PS = np.zeros((16, EVW), dtype=np.float32)
for _o in range(OUT_DIM):
    _PC[_O2P[_o], _o] = 1.0
    _PS[_O2D[_o], _o] = 1.0

_TILE8 = np.tile(np.eye(MID, dtype=np.float32), (1, NPATH))
_SUMR = np.zeros((MID * NPATH, NPATH), dtype=np.float32)
for _p in range(NPATH):
    _SUMR[_p * MID:(_p + 1) * MID, _p] = 1.0

_S3, _S15, _S5 = np.sqrt(3.0), np.sqrt(15.0), np.sqrt(5.0)
_S105, _S7 = np.sqrt(105.0), np.sqrt(7.0)
_S35_8, _S21_8 = np.sqrt(35.0 / 8.0), np.sqrt(21.0 / 8.0)


def _table_body(x_ref, mol_ref, pos_ref, zt_ref, mt_ref, t_ref):
    nb = x_ref.shape[0]
    ioz = lax.broadcasted_iota(jnp.int32, (nb, 128), 1)
    oh = (x_ref[...] == ioz).astype(jnp.float32)
    iom = lax.broadcasted_iota(jnp.int32, (nb, 8), 1)
    ohm = (mol_ref[...] == iom).astype(jnp.float32)
    h = (jnp.dot(oh, zt_ref[...], preferred_element_type=jnp.float32)
         + jnp.dot(ohm, mt_ref[...], preferred_element_type=jnp.float32))
    pos = pos_ref[...]
    zpad = jnp.zeros((nb, TW - 67), dtype=jnp.float32)
    t_ref[...] = jnp.concatenate([h, pos, zpad], axis=1)


def _gather_body(t_hbm, pos_hbm, src_hbm, dstf_hbm, g1_hbm,
                 idxs_v, idxd_v, pos_v, buf0_v, buf1_v, sem0, sem1):
    sid = lax.axis_index("s")
    wid = lax.axis_index("c") * NS + sid
    base = wid * EPWP
    pltpu.sync_copy(pos_hbm, pos_v)
    pltpu.sync_copy(src_hbm.at[wid], idxs_v)
    pltpu.sync_copy(dstf_hbm.at[wid], idxd_v)

    lrow = lax.broadcasted_iota(jnp.int32, (16,), 0)

    def merge_and_out(j, buf):
        for i in range(CH // 16):
            rows = idxd_v[pl.ds(j * CH + i * 16, 16)]
            for c in range(3):
                vals = plsc.load_gather(pos_v, [rows * 3 + c])
                plsc.store_scatter(
                    buf, [lrow + i * 16,
                          jnp.full((16,), 68 + c, jnp.int32)], vals)
        pltpu.sync_copy(buf, g1_hbm.at[pl.ds(base + j * CH, CH)])

    def start(j, buf, sem):
        return pltpu.async_copy(t_hbm.at[idxs_v.at[j]], buf, sem)

    start(0, buf0_v, sem0)

    def body(j2, carry):
        j = j2 * 2
        c1 = start(j + 1, buf1_v, sem1)
        pltpu.make_async_copy(t_hbm.at[idxs_v.at[j]], buf0_v, sem0).wait()
        merge_and_out(j, buf0_v)
        start(j + 2, buf0_v, sem0)
        c1.wait()
        merge_and_out(j + 1, buf1_v)
        return carry

    lax.fori_loop(0, NCHUNK // 2 - 1, body, 0)
    j = NCHUNK - 2
    c1 = start(j + 1, buf1_v, sem1)
    pltpu.make_async_copy(t_hbm.at[idxs_v.at[j]], buf0_v, sem0).wait()
    merge_and_out(j, buf0_v)
    c1.wait()
    merge_and_out(j + 1, buf1_v)


def _edge_body(g1_ref, w1_ref, w2_ref, t8_ref, sr_ref, pc_ref, ps_ref,
               ev_ref):
    eb = g1_ref.shape[0]
    h_src = g1_ref[:, :NODE_DIM]
    vec = g1_ref[:, 68:71] - g1_ref[:, 64:67]
    elen = jnp.sqrt(jnp.sum(vec * vec, axis=1, keepdims=True))
    unit = vec / (elen + 1e-12)
    ux, uy, uz = unit[:, 0:1], unit[:, 1:2], unit[:, 2:3]

    xx, yy, zz = ux * ux, uy * uy, uz * uz
    one = jnp.ones_like(ux)
    sh = jnp.concatenate([
        one,
        _S3 * ux, _S3 * uy, _S3 * uz,
        _S15 * ux * uy,
        _S15 * uy * uz,
        (_S5 / 2.0) * (3.0 * zz - 1.0),
        _S15 * ux * uz,
        (_S15 / 2.0) * (xx - yy),
        _S35_8 * uy * (3.0 * xx - yy),
        _S105 * ux * uy * uz,
        _S21_8 * uy * (5.0 * zz - 1.0),
        (_S7 / 2.0) * uz * (5.0 * zz - 3.0),
        _S21_8 * ux * (5.0 * zz - 1.0),
        (_S105 / 2.0) * uz * (xx - yy),
        _S35_8 * ux * (xx - 3.0 * yy),
    ], axis=1)

    lane = lax.broadcasted_iota(jnp.int32, (eb, 16), 1)
    dif = elen / STEP - lane.astype(jnp.float32) - 1.0

    def sus(t):
        return jnp.where(t > 0, jnp.exp(-1.0 / jnp.where(t > 0, t, 1.0)), 0.0)

    f = sus(dif + 1.0) * sus(1.0 - dif)
    f = jnp.where(lane < NUM_BASIS, f, 0.0)

    z1 = jnp.dot(f, w1_ref[...], preferred_element_type=jnp.float32)
    h1 = (z1 / (1.0 + jnp.exp(-z1))) * SILU_CST

    a = jnp.dot(h_src, w2_ref[...], preferred_element_type=jnp.float32)
    htile = jnp.dot(h1, t8_ref[...], preferred_element_type=jnp.float32)
    coeff = jnp.dot(a * htile, sr_ref[...],
                    preferred_element_type=jnp.float32)
    ev = (jnp.dot(coeff, pc_ref[...], preferred_element_type=jnp.float32)
          * jnp.dot(sh, ps_ref[...], preferred_element_type=jnp.float32))
    row = (pl.program_id(0) * eb
           + lax.broadcasted_iota(jnp.int32, (eb, 1), 0))
    valid = jnp.mod(row, EPWP) < EPW
    ev = jnp.where(valid, ev, 0.0)
    zpad8 = jnp.zeros((eb, 8), jnp.float32)
    for cg in range(NCG):
        ev_ref[cg] = jnp.concatenate([ev[:, cg * 8:(cg + 1) * 8], zpad8],
                                     axis=1)


def _scatter_body(ev_hbm, idx_hbm, zero_hbm, nv_hbm, idxb_v, evb_v, acc_v):
    wid = lax.axis_index("c") * NS + lax.axis_index("s")
    cg = wid // NEG
    eg = wid % NEG
    ebase = cg * E_PAD + eg * ECG

    pltpu.sync_copy(zero_hbm, acc_v)

    def chunk(j, carry):
        pltpu.sync_copy(ev_hbm.at[pl.ds(ebase + j * CH, CH)], evb_v)
        pltpu.sync_copy(idx_hbm.at[pl.ds(eg * ECG + j * CH, CH)], idxb_v)

        def row(r4, c2):
            for u in range(4):
                r = r4 * 4 + u
                plsc.addupdate_scatter(acc_v, [idxb_v[r]], evb_v[r])
            return c2

        lax.fori_loop(0, CH // 4, row, 0)
        return carry

    lax.fori_loop(0, NCH2, chunk, 0)
    pltpu.sync_copy(acc_v, nv_hbm.at[pl.ds(wid * NPAD * 8, NPAD * 8)])


def _ln_body(nv_ref, g_ref, b_ref, out_ref):
    nb = nv_ref.shape[1]
    parts = []
    for cg in range(NCG):
        p = nv_ref[cg * NEG]
        for eg in range(1, NEG):
            p = p + nv_ref[cg * NEG + eg]
        parts.append(p)
    v = jnp.concatenate(parts, axis=1)
    mu = jnp.sum(v, axis=1, keepdims=True) * (1.0 / OUT_DIM)
    lane = lax.broadcasted_iota(jnp.int32, (nb, EVW), 1)
    d = jnp.where(lane < OUT_DIM, v - mu, 0.0)
    var = jnp.sum(d * d, axis=1, keepdims=True) * (1.0 / OUT_DIM)
    res = d * lax.rsqrt(var + 1e-5) * g_ref[...] + b_ref[...]
    out_ref[...] = res[:, :OUT_DIM]


@jax.jit
def _run(x, pos, mol_id, edge_src, edge_dst, zt_pad, mt_pad, w1p, w2rt,
         t8, sr, pc, ps, gamma_p, beta_p, zeros_nv):
    nb = 1000
    grid_a = N_NODES // nb
    tbl = pl.pallas_call(
        _table_body,
        grid=(grid_a,),
        in_specs=[
            pl.BlockSpec((nb, 1), lambda i: (i, 0)),
            pl.BlockSpec((nb, 1), lambda i: (i, 0)),
            pl.BlockSpec((nb, 3), lambda i: (i, 0)),
            pl.BlockSpec((128, NODE_DIM), lambda i: (0, 0)),
            pl.BlockSpec((8, NODE_DIM), lambda i: (0, 0)),
        ],
        out_specs=pl.BlockSpec((nb, TW), lambda i: (i, 0)),
        out_shape=jax.ShapeDtypeStruct((N_NODES, TW), jnp.float32),
    )(x.reshape(N_NODES, 1), mol_id.reshape(N_NODES, 1), pos, zt_pad, mt_pad)

    src3 = jnp.pad(edge_src.reshape(NW, EPW),
                   ((0, 0), (0, EPWP - EPW))).reshape(NW, NCHUNK, CH)
    dst3 = jnp.pad(edge_dst.reshape(NW, EPW),
                   ((0, 0), (0, EPWP - EPW))).reshape(NW, NCHUNK, CH)

    pos4 = jnp.pad(pos, ((0, NPAD - N_NODES), (0, 0))).reshape(-1)
    dstf = jnp.pad(edge_dst.reshape(NW, EPW), ((0, 0), (0, EPWP - EPW)))
    g1 = pl.kernel(
        _gather_body,
        out_type=jax.ShapeDtypeStruct((E_PAD, TW), jnp.float32),
        mesh=plsc.VectorSubcoreMesh(core_axis_name="c", subcore_axis_name="s"),
        compiler_params=pltpu.CompilerParams(needs_layout_passes=False),
        scratch_types=[
            pltpu.VMEM((NCHUNK, CH), jnp.int32),
            pltpu.VMEM((EPWP,), jnp.int32),
            pltpu.VMEM((3 * NPAD,), jnp.float32),
            pltpu.VMEM((CH, TW), jnp.float32),
            pltpu.VMEM((CH, TW), jnp.float32),
            pltpu.SemaphoreType.DMA,
            pltpu.SemaphoreType.DMA,
        ],
    )(tbl, pos4, src3, dstf)

    eb = 640
    grid_c = E_PAD // eb
    ev = pl.pallas_call(
        _edge_body,
        grid=(grid_c,),
        in_specs=[
            pl.BlockSpec((eb, TW), lambda i: (i, 0)),
            pl.BlockSpec((16, MID), lambda i: (0, 0)),
            pl.BlockSpec((NODE_DIM, MID * NPATH), lambda i: (0, 0)),
            pl.BlockSpec((MID, MID * NPATH), lambda i: (0, 0)),
            pl.BlockSpec((MID * NPATH, NPATH), lambda i: (0, 0)),
            pl.BlockSpec((NPATH, EVW), lambda i: (0, 0)),
            pl.BlockSpec((16, EVW), lambda i: (0, 0)),
        ],
        out_specs=pl.BlockSpec((NCG, eb, 16), lambda i: (0, i, 0)),
        out_shape=jax.ShapeDtypeStruct((NCG, E_PAD, 16), jnp.float32),
    )(g1, w1p, w2rt, t8, sr, pc, ps)
    ev = ev.reshape(NCG * E_PAD, 16)

    dstp = jnp.pad(edge_dst.reshape(NW, EPW),
                   ((0, 0), (0, EPWP - EPW))).reshape(E_PAD, 1)
    cpat = jnp.minimum(jnp.arange(16, dtype=jnp.int32), 7)[None, :]
    idxe = dstp * 8 + cpat
    nv2 = pl.kernel(
        _scatter_body,
        out_type=jax.ShapeDtypeStruct((NW * NPAD * 8,), jnp.float32),
        mesh=plsc.VectorSubcoreMesh(core_axis_name="c", subcore_axis_name="s"),
        compiler_params=pltpu.CompilerParams(needs_layout_passes=False),
        scratch_types=[
            pltpu.VMEM((CH, 16), jnp.int32),
            pltpu.VMEM((CH, 16), jnp.float32),
            pltpu.VMEM((NPAD * 8,), jnp.float32),
        ],
    )(ev, idxe, zeros_nv)
    nv2 = nv2.reshape(NW, NPAD, 8)

    out = pl.pallas_call(
        _ln_body,
        grid=(grid_a,),
        in_specs=[
            pl.BlockSpec((NW, nb, 8), lambda i: (0, i, 0)),
            pl.BlockSpec((1, EVW), lambda i: (0, 0)),
            pl.BlockSpec((1, EVW), lambda i: (0, 0)),
        ],
        out_specs=pl.BlockSpec((nb, OUT_DIM), lambda i: (i, 0)),
        out_shape=jax.ShapeDtypeStruct((N_NODES, OUT_DIM), jnp.float32),
    )(nv2, gamma_p, beta_p)
    return out


def kernel(x, pos, mol_id, edge_src, edge_dst, z_table, mol_table, W1, W2,
           gamma, beta):
    x = x.astype(jnp.int32)
    mol_id = mol_id.astype(jnp.int32)
    edge_src = edge_src.astype(jnp.int32)
    edge_dst = edge_dst.astype(jnp.int32)

    zt_pad = jnp.zeros((128, NODE_DIM), jnp.float32)
    zt_pad = zt_pad.at[:NUM_ELEMENTS, :Z_DIM].set(z_table)
    mt_pad = jnp.zeros((8, NODE_DIM), jnp.float32)
    mt_pad = mt_pad.at[:2, Z_DIM:].set(mol_table)

    w1p = jnp.zeros((16, MID), jnp.float32).at[:NUM_BASIS].set(W1 * EMB_CST)

    scale = 1.0 / (np.sqrt(float(MID)) * np.sqrt(float(NODE_DIM))
                   * np.sqrt(float(N_EDGES) / float(N_NODES)))
    w2rt = W2.T[_COLIDX.reshape(-1)].reshape(NODE_DIM, NPATH, MID)
    w2rt = (w2rt * scale).reshape(NODE_DIM, NPATH * MID)

    gamma_p = jnp.zeros((1, EVW), jnp.float32).at[0, :OUT_DIM].set(gamma)
    beta_p = jnp.zeros((1, EVW), jnp.float32).at[0, :OUT_DIM].set(beta)
    zeros_nv = jnp.zeros((NPAD * 8,), jnp.float32)

    return _run(x, pos, mol_id, edge_src, edge_dst, zt_pad, mt_pad, w1p,
                w2rt, jnp.asarray(_TILE8), jnp.asarray(_SUMR),
                jnp.asarray(_PC), jnp.asarray(_PS), gamma_p, beta_p,
                zeros_nv)


# speedup vs baseline: 1.6407x; 1.2149x over previous
import numpy as np
import jax
import jax.numpy as jnp
from jax import lax
from jax.experimental import pallas as pl
from jax.experimental.pallas import tpu as pltpu
from jax.experimental.pallas import tpu_sc as plsc

N_NODES = 10000
N_EDGES = 160000
NUM_ELEMENTS = 100
Z_DIM = 48
MOL_DIM = 16
NODE_DIM = 64
NUM_BASIS = 10
MID = 32
MAX_RADIUS = 2.0
MULS = [4, 2, 1, 1]
DIMS = [1, 3, 5, 7]
NPATH = 8
OUT_DIM = 22
EVW = 32
TW = 128

NC, NS = 2, 16
NW = NC * NS
EPW = N_EDGES // NW
CH = 128
NCHUNK = 40
EPWP = CH * NCHUNK
E_PAD = EPWP * NW
NEG = 8
NCG = 4
ECG = E_PAD // NEG
NCH2 = ECG // CH
CHS = 64
NCH3 = ECG // CHS
NPAD = 10112
RPT = NPAD // NS

_t = np.linspace(-12.0, 12.0, 480001)
_pdf = np.exp(-0.5 * _t * _t) / np.sqrt(2.0 * np.pi)
_s = _t / (1.0 + np.exp(-_t))
_trapz = getattr(np, "trapz", None) or np.trapezoid
SILU_CST = float(1.0 / np.sqrt(_trapz(_s * _s * _pdf, _t)))

STEP = MAX_RADIUS / (NUM_BASIS + 1)
EMB_CST = 1.14136 * float(np.exp(2.0))

_OFFS = [0, 256, 384, 448]
_PATH = []
for _l, _m in enumerate(MULS):
    for _j in range(_m):
        _PATH.append((_l, _OFFS[_l], _m, _j))

_COLIDX = np.zeros((NODE_DIM, NPATH), dtype=np.int32)
for _p, (_l, _off, _m, _j) in enumerate(_PATH):
    for _u in range(NODE_DIM):
        _COLIDX[_u, _p] = _off + _u * _m + _j

_SH_OFF = [0, 1, 4, 9]
_O2P = np.zeros(OUT_DIM, dtype=np.int32)
_O2D = np.zeros(OUT_DIM, dtype=np.int32)
_o = 0
for _p, (_l, _off, _m, _j) in enumerate(_PATH):
    for _c in range(DIMS[_l]):
        _O2P[_o] = _p
        _O2D[_o] = _SH_OFF[_l] + _c
        _o += 1

_PC = np.zeros((NPATH, EVW), dtype=np.float32)
_PS = np.zeros((16, EVW), dtype=np.float32)
for _o in range(OUT_DIM):
    _PC[_O2P[_o], _o] = 1.0
    _PS[_O2D[_o], _o] = 1.0

_TILE8 = np.tile(np.eye(MID, dtype=np.float32), (1, NPATH))
_SUMR = np.zeros((MID * NPATH, NPATH), dtype=np.float32)
for _p in range(NPATH):
    _SUMR[_p * MID:(_p + 1) * MID, _p] = 1.0

_S3, _S15, _S5 = np.sqrt(3.0), np.sqrt(15.0), np.sqrt(5.0)
_S105, _S7 = np.sqrt(105.0), np.sqrt(7.0)
_S35_8, _S21_8 = np.sqrt(35.0 / 8.0), np.sqrt(21.0 / 8.0)


def _table_body(x_ref, mol_ref, pos_ref, zt_ref, mt_ref, t_ref):
    nb = x_ref.shape[0]
    ioz = lax.broadcasted_iota(jnp.int32, (nb, 128), 1)
    oh = (x_ref[...] == ioz).astype(jnp.float32)
    iom = lax.broadcasted_iota(jnp.int32, (nb, 8), 1)
    ohm = (mol_ref[...] == iom).astype(jnp.float32)
    h = (jnp.dot(oh, zt_ref[...], preferred_element_type=jnp.float32)
         + jnp.dot(ohm, mt_ref[...], preferred_element_type=jnp.float32))
    pos = pos_ref[...]
    zpad = jnp.zeros((nb, TW - 67), dtype=jnp.float32)
    t_ref[...] = jnp.concatenate([h, pos, zpad], axis=1)


def _gather_body(t_hbm, pos_hbm, src_hbm, dstf_hbm, g1_hbm,
                 idxs_v, idxd_v, pos_v, buf0_v, buf1_v, sem0, sem1):
    sid = lax.axis_index("s")
    wid = lax.axis_index("c") * NS + sid
    base = wid * EPWP
    pltpu.sync_copy(pos_hbm, pos_v)
    pltpu.sync_copy(src_hbm.at[wid], idxs_v)
    pltpu.sync_copy(dstf_hbm.at[wid], idxd_v)

    lrow = lax.broadcasted_iota(jnp.int32, (16,), 0)

    def merge_and_out(j, buf):
        for i in range(CH // 16):
            rows = idxd_v[pl.ds(j * CH + i * 16, 16)]
            for c in range(3):
                vals = plsc.load_gather(pos_v, [rows * 3 + c])
                plsc.store_scatter(
                    buf, [lrow + i * 16,
                          jnp.full((16,), 68 + c, jnp.int32)], vals)
        pltpu.sync_copy(buf, g1_hbm.at[pl.ds(base + j * CH, CH)])

    def start(j, buf, sem):
        return pltpu.async_copy(t_hbm.at[idxs_v.at[j]], buf, sem)

    start(0, buf0_v, sem0)

    def body(j2, carry):
        j = j2 * 2
        c1 = start(j + 1, buf1_v, sem1)
        pltpu.make_async_copy(t_hbm.at[idxs_v.at[j]], buf0_v, sem0).wait()
        merge_and_out(j, buf0_v)
        start(j + 2, buf0_v, sem0)
        c1.wait()
        merge_and_out(j + 1, buf1_v)
        return carry

    lax.fori_loop(0, NCHUNK // 2 - 1, body, 0)
    j = NCHUNK - 2
    c1 = start(j + 1, buf1_v, sem1)
    pltpu.make_async_copy(t_hbm.at[idxs_v.at[j]], buf0_v, sem0).wait()
    merge_and_out(j, buf0_v)
    c1.wait()
    merge_and_out(j + 1, buf1_v)


def _edge_body(g1_ref, w1_ref, w2_ref, t8_ref, sr_ref, pc_ref, ps_ref,
               ev_ref):
    eb = g1_ref.shape[0]
    h_src = g1_ref[:, :NODE_DIM]
    vec = g1_ref[:, 68:71] - g1_ref[:, 64:67]
    elen = jnp.sqrt(jnp.sum(vec * vec, axis=1, keepdims=True))
    unit = vec / (elen + 1e-12)
    ux, uy, uz = unit[:, 0:1], unit[:, 1:2], unit[:, 2:3]

    xx, yy, zz = ux * ux, uy * uy, uz * uz
    one = jnp.ones_like(ux)
    sh = jnp.concatenate([
        one,
        _S3 * ux, _S3 * uy, _S3 * uz,
        _S15 * ux * uy,
        _S15 * uy * uz,
        (_S5 / 2.0) * (3.0 * zz - 1.0),
        _S15 * ux * uz,
        (_S15 / 2.0) * (xx - yy),
        _S35_8 * uy * (3.0 * xx - yy),
        _S105 * ux * uy * uz,
        _S21_8 * uy * (5.0 * zz - 1.0),
        (_S7 / 2.0) * uz * (5.0 * zz - 3.0),
        _S21_8 * ux * (5.0 * zz - 1.0),
        (_S105 / 2.0) * uz * (xx - yy),
        _S35_8 * ux * (xx - 3.0 * yy),
    ], axis=1)

    lane = lax.broadcasted_iota(jnp.int32, (eb, 16), 1)
    dif = elen / STEP - lane.astype(jnp.float32) - 1.0

    def sus(t):
        return jnp.where(t > 0, jnp.exp(-1.0 / jnp.where(t > 0, t, 1.0)), 0.0)

    f = sus(dif + 1.0) * sus(1.0 - dif)
    f = jnp.where(lane < NUM_BASIS, f, 0.0)

    z1 = jnp.dot(f, w1_ref[...], preferred_element_type=jnp.float32)
    h1 = (z1 / (1.0 + jnp.exp(-z1))) * SILU_CST

    a = jnp.dot(h_src, w2_ref[...], preferred_element_type=jnp.float32)
    htile = jnp.dot(h1, t8_ref[...], preferred_element_type=jnp.float32)
    coeff = jnp.dot(a * htile, sr_ref[...],
                    preferred_element_type=jnp.float32)
    ev = (jnp.dot(coeff, pc_ref[...], preferred_element_type=jnp.float32)
          * jnp.dot(sh, ps_ref[...], preferred_element_type=jnp.float32))
    row = (pl.program_id(0) * eb
           + lax.broadcasted_iota(jnp.int32, (eb, 1), 0))
    valid = jnp.mod(row, EPWP) < EPW
    ev = jnp.where(valid, ev, 0.0)
    zpad8 = jnp.zeros((eb, 8), jnp.float32)
    for cg in range(NCG):
        ev_ref[cg] = jnp.concatenate([ev[:, cg * 8:(cg + 1) * 8], zpad8],
                                     axis=1)


def _scatter_body(ev_hbm, idx_hbm, zero_hbm, nv_hbm, idxb0_v, evb0_v,
                  idxb1_v, evb1_v, acc_v, se0, si0, se1, si1):
    wid = lax.axis_index("c") * NS + lax.axis_index("s")
    cg = wid // NEG
    eg = wid % NEG
    ebase = cg * E_PAD + eg * ECG

    pltpu.sync_copy(zero_hbm, acc_v)

    def start(j, evb, idxb, se, si):
        pltpu.async_copy(ev_hbm.at[pl.ds(ebase + j * CHS, CHS)], evb, se)
        pltpu.async_copy(idx_hbm.at[pl.ds(eg * ECG + j * CHS, CHS)], idxb, si)

    def drain(j, evb, idxb, se, si):
        pltpu.make_async_copy(ev_hbm.at[pl.ds(ebase + j * CHS, CHS)],
                              evb, se).wait()
        pltpu.make_async_copy(idx_hbm.at[pl.ds(eg * ECG + j * CHS, CHS)],
                              idxb, si).wait()

        def row(r4, c2):
            for u in range(4):
                r = r4 * 4 + u
                plsc.addupdate_scatter(acc_v, [idxb[r]], evb[r])
            return c2

        lax.fori_loop(0, CHS // 4, row, 0)

    start(0, evb0_v, idxb0_v, se0, si0)

    def chunk2(j2, carry):
        j = j2 * 2
        start(j + 1, evb1_v, idxb1_v, se1, si1)
        drain(j, evb0_v, idxb0_v, se0, si0)
        start(j + 2, evb0_v, idxb0_v, se0, si0)
        drain(j + 1, evb1_v, idxb1_v, se1, si1)
        return carry

    lax.fori_loop(0, NCH3 // 2 - 1, chunk2, 0)
    j = NCH3 - 2
    start(j + 1, evb1_v, idxb1_v, se1, si1)
    drain(j, evb0_v, idxb0_v, se0, si0)
    drain(j + 1, evb1_v, idxb1_v, se1, si1)
    pltpu.sync_copy(acc_v, nv_hbm.at[pl.ds(wid * NPAD * 8, NPAD * 8)])


def _ln_body(nv_ref, g_ref, b_ref, out_ref):
    nb = nv_ref.shape[1]
    parts = []
    for cg in range(NCG):
        p = nv_ref[cg * NEG]
        for eg in range(1, NEG):
            p = p + nv_ref[cg * NEG + eg]
        parts.append(p)
    v = jnp.concatenate(parts, axis=1)
    mu = jnp.sum(v, axis=1, keepdims=True) * (1.0 / OUT_DIM)
    lane = lax.broadcasted_iota(jnp.int32, (nb, EVW), 1)
    d = jnp.where(lane < OUT_DIM, v - mu, 0.0)
    var = jnp.sum(d * d, axis=1, keepdims=True) * (1.0 / OUT_DIM)
    res = d * lax.rsqrt(var + 1e-5) * g_ref[...] + b_ref[...]
    out_ref[...] = res[:, :OUT_DIM]


@jax.jit
def _run(x, pos, mol_id, edge_src, edge_dst, zt_pad, mt_pad, w1p, w2rt,
         t8, sr, pc, ps, gamma_p, beta_p, zeros_nv):
    nb = 1000
    grid_a = N_NODES // nb
    tbl = pl.pallas_call(
        _table_body,
        grid=(grid_a,),
        in_specs=[
            pl.BlockSpec((nb, 1), lambda i: (i, 0)),
            pl.BlockSpec((nb, 1), lambda i: (i, 0)),
            pl.BlockSpec((nb, 3), lambda i: (i, 0)),
            pl.BlockSpec((128, NODE_DIM), lambda i: (0, 0)),
            pl.BlockSpec((8, NODE_DIM), lambda i: (0, 0)),
        ],
        out_specs=pl.BlockSpec((nb, TW), lambda i: (i, 0)),
        out_shape=jax.ShapeDtypeStruct((N_NODES, TW), jnp.float32),
    )(x.reshape(N_NODES, 1), mol_id.reshape(N_NODES, 1), pos, zt_pad, mt_pad)

    src3 = jnp.pad(edge_src.reshape(NW, EPW),
                   ((0, 0), (0, EPWP - EPW))).reshape(NW, NCHUNK, CH)
    dst3 = jnp.pad(edge_dst.reshape(NW, EPW),
                   ((0, 0), (0, EPWP - EPW))).reshape(NW, NCHUNK, CH)

    pos4 = jnp.pad(pos, ((0, NPAD - N_NODES), (0, 0))).reshape(-1)
    dstf = jnp.pad(edge_dst.reshape(NW, EPW), ((0, 0), (0, EPWP - EPW)))
    g1 = pl.kernel(
        _gather_body,
        out_type=jax.ShapeDtypeStruct((E_PAD, TW), jnp.float32),
        mesh=plsc.VectorSubcoreMesh(core_axis_name="c", subcore_axis_name="s"),
        compiler_params=pltpu.CompilerParams(needs_layout_passes=False),
        scratch_types=[
            pltpu.VMEM((NCHUNK, CH), jnp.int32),
            pltpu.VMEM((EPWP,), jnp.int32),
            pltpu.VMEM((3 * NPAD,), jnp.float32),
            pltpu.VMEM((CH, TW), jnp.float32),
            pltpu.VMEM((CH, TW), jnp.float32),
            pltpu.SemaphoreType.DMA,
            pltpu.SemaphoreType.DMA,
        ],
    )(tbl, pos4, src3, dstf)

    eb = 640
    grid_c = E_PAD // eb
    ev = pl.pallas_call(
        _edge_body,
        grid=(grid_c,),
        in_specs=[
            pl.BlockSpec((eb, TW), lambda i: (i, 0)),
            pl.BlockSpec((16, MID), lambda i: (0, 0)),
            pl.BlockSpec((NODE_DIM, MID * NPATH), lambda i: (0, 0)),
            pl.BlockSpec((MID, MID * NPATH), lambda i: (0, 0)),
            pl.BlockSpec((MID * NPATH, NPATH), lambda i: (0, 0)),
            pl.BlockSpec((NPATH, EVW), lambda i: (0, 0)),
            pl.BlockSpec((16, EVW), lambda i: (0, 0)),
        ],
        out_specs=pl.BlockSpec((NCG, eb, 16), lambda i: (0, i, 0)),
        out_shape=jax.ShapeDtypeStruct((NCG, E_PAD, 16), jnp.float32),
    )(g1, w1p, w2rt, t8, sr, pc, ps)
    ev = ev.reshape(NCG * E_PAD, 16)

    dstp = jnp.pad(edge_dst.reshape(NW, EPW),
                   ((0, 0), (0, EPWP - EPW))).reshape(E_PAD, 1)
    cpat = jnp.minimum(jnp.arange(16, dtype=jnp.int32), 7)[None, :]
    idxe = dstp * 8 + cpat
    nv2 = pl.kernel(
        _scatter_body,
        out_type=jax.ShapeDtypeStruct((NW * NPAD * 8,), jnp.float32),
        mesh=plsc.VectorSubcoreMesh(core_axis_name="c", subcore_axis_name="s"),
        compiler_params=pltpu.CompilerParams(needs_layout_passes=False),
        scratch_types=[
            pltpu.VMEM((CHS, 16), jnp.int32),
            pltpu.VMEM((CHS, 16), jnp.float32),
            pltpu.VMEM((CHS, 16), jnp.int32),
            pltpu.VMEM((CHS, 16), jnp.float32),
            pltpu.VMEM((NPAD * 8,), jnp.float32),
            pltpu.SemaphoreType.DMA,
            pltpu.SemaphoreType.DMA,
            pltpu.SemaphoreType.DMA,
            pltpu.SemaphoreType.DMA,
        ],
    )(ev, idxe, zeros_nv)
    nv2 = nv2.reshape(NW, NPAD, 8)

    out = pl.pallas_call(
        _ln_body,
        grid=(grid_a,),
        in_specs=[
            pl.BlockSpec((NW, nb, 8), lambda i: (0, i, 0)),
            pl.BlockSpec((1, EVW), lambda i: (0, 0)),
            pl.BlockSpec((1, EVW), lambda i: (0, 0)),
        ],
        out_specs=pl.BlockSpec((nb, OUT_DIM), lambda i: (i, 0)),
        out_shape=jax.ShapeDtypeStruct((N_NODES, OUT_DIM), jnp.float32),
    )(nv2, gamma_p, beta_p)
    return out


def kernel(x, pos, mol_id, edge_src, edge_dst, z_table, mol_table, W1, W2,
           gamma, beta):
    x = x.astype(jnp.int32)
    mol_id = mol_id.astype(jnp.int32)
    edge_src = edge_src.astype(jnp.int32)
    edge_dst = edge_dst.astype(jnp.int32)

    zt_pad = jnp.zeros((128, NODE_DIM), jnp.float32)
    zt_pad = zt_pad.at[:NUM_ELEMENTS, :Z_DIM].set(z_table)
    mt_pad = jnp.zeros((8, NODE_DIM), jnp.float32)
    mt_pad = mt_pad.at[:2, Z_DIM:].set(mol_table)

    w1p = jnp.zeros((16, MID), jnp.float32).at[:NUM_BASIS].set(W1 * EMB_CST)

    scale = 1.0 / (np.sqrt(float(MID)) * np.sqrt(float(NODE_DIM))
                   * np.sqrt(float(N_EDGES) / float(N_NODES)))
    w2rt = W2.T[_COLIDX.reshape(-1)].reshape(NODE_DIM, NPATH, MID)
    w2rt = (w2rt * scale).reshape(NODE_DIM, NPATH * MID)

    gamma_p = jnp.zeros((1, EVW), jnp.float32).at[0, :OUT_DIM].set(gamma)
    beta_p = jnp.zeros((1, EVW), jnp.float32).at[0, :OUT_DIM].set(beta)
    zeros_nv = jnp.zeros((NPAD * 8,), jnp.float32)

    return _run(x, pos, mol_id, edge_src, edge_dst, zt_pad, mt_pad, w1p,
                w2rt, jnp.asarray(_TILE8), jnp.asarray(_SUMR),
                jnp.asarray(_PC), jnp.asarray(_PS), gamma_p, beta_p,
                zeros_nv)
